# baseline (speedup 1.0000x reference)
import jax
import jax.numpy as jnp
from jax.experimental import pallas as pl
from jax.experimental.pallas import tpu as pltpu

_H = 8
_EPS = 1e-5
_BN = 512


def kernel(class_feature, features, q_gamma, q_beta, Wq, kv_gamma, kv_beta, Wkv, proj_W, proj_b):
    B, N, C = features.shape
    H = _H
    D = C // H
    BN = _BN
    NC = N // BN
    scale = C ** -0.5

    cf2 = class_feature.reshape(1, C)
    qg2 = q_gamma.reshape(1, C)
    qb2 = q_beta.reshape(1, C)
    kvg2 = kv_gamma.reshape(1, C)
    kvb2 = kv_beta.reshape(1, C)
    pb2 = proj_b.reshape(1, C)
    Wk = Wkv[:C]
    Wv = Wkv[C:]

    def body(x_ref, cf_ref, qg_ref, qb_ref, wq_ref, wk_ref, wv_ref,
             kvg_ref, kvb_ref, pw_ref, pb_ref, o_ref,
             waug, acc, m_s, d_s, t_s, g0_s, g1_s):
        nc = pl.program_id(1)

        head_mask = jnp.where(
            jax.lax.broadcasted_iota(jnp.int32, (H, C), 1) // D
            == jax.lax.broadcasted_iota(jnp.int32, (H, C), 0),
            1.0, 0.0).astype(jnp.float32)

        @pl.when(nc == 0)
        def _prep():
            cf = cf_ref[...]
            mu = jnp.mean(cf, axis=1, keepdims=True)
            xc = cf - mu
            var = jnp.mean(xc * xc, axis=1, keepdims=True)
            ln = xc * jax.lax.rsqrt(var + _EPS) * qg_ref[...] + qb_ref[...]
            q = jax.lax.dot_general(ln, wq_ref[...], (((1,), (1,)), ((), ())),
                                    preferred_element_type=jnp.float32)
            qs = q * scale
            A = jnp.broadcast_to(qs, (H, C)) * head_mask
            wl = jax.lax.dot_general(A, wk_ref[...], (((1,), (0,)), ((), ())),
                                     preferred_element_type=jnp.float32)
            wlg = wl * kvg_ref[...]
            waug[0:H, :] = wlg.astype(jnp.bfloat16)
            row = jax.lax.broadcasted_iota(jnp.int32, (8, C), 0)
            waug[H:H + 8, :] = jnp.where(row == 0, 1.0, 0.0).astype(jnp.bfloat16)
            g1_s[...] = jnp.sum(wlg, axis=1, keepdims=True)
            g0_s[...] = jnp.sum(wl * kvb_ref[...], axis=1, keepdims=True)
            m_s[...] = jnp.full((H, 1), -1e30, jnp.float32)
            d_s[...] = jnp.zeros((H, 1), jnp.float32)
            t_s[...] = jnp.zeros((H, 1), jnp.float32)
            acc[...] = jnp.zeros((H, C), jnp.float32)

        x = x_ref[0]
        xb = x.astype(jnp.bfloat16)
        m1 = jax.lax.dot_general(waug[...], xb, (((1,), (1,)), ((), ())),
                                 preferred_element_type=jnp.float32)
        ones_row = jnp.ones((1, C), jnp.bfloat16)
        m2 = jax.lax.dot_general(ones_row, xb * xb, (((1,), (1,)), ((), ())),
                                 preferred_element_type=jnp.float32)

        inv_c = 1.0 / C
        mu_r = m1[H:H + 1, :] * inv_c
        var_r = m2 * inv_c - mu_r * mu_r
        s_r = jax.lax.rsqrt(var_r + _EPS)
        logits = s_r * (m1[0:H, :] - mu_r * g1_s[...]) + g0_s[...]

        m_prev = m_s[...]
        lm = jnp.max(logits, axis=1, keepdims=True)
        m_new = jnp.maximum(m_prev, lm)
        alpha = jnp.exp(m_prev - m_new)
        p = jnp.exp(logits - m_new)
        ps = p * s_r
        d_s[...] = d_s[...] * alpha + jnp.sum(p, axis=1, keepdims=True)
        t_s[...] = t_s[...] * alpha + jnp.sum(ps * mu_r, axis=1, keepdims=True)
        m_s[...] = m_new
        acc[...] = acc[...] * alpha + jax.lax.dot_general(
            ps.astype(jnp.bfloat16), xb, (((1,), (0,)), ((), ())),
            preferred_element_type=jnp.float32)

        @pl.when(nc == NC - 1)
        def _fin():
            dinv = 1.0 / d_s[...]
            S = kvg_ref[...] * (acc[...] * dinv - t_s[...] * dinv) + kvb_ref[...]
            R = jax.lax.dot_general(S, wv_ref[...], (((1,), (1,)), ((), ())),
                                    preferred_element_type=jnp.float32)
            agg = jnp.sum(R * head_mask, axis=0, keepdims=True)
            o = jax.lax.dot_general(agg, pw_ref[...], (((1,), (1,)), ((), ())),
                                    preferred_element_type=jnp.float32)
            o_ref[...] = (cf_ref[...] + o + pb_ref[...]).reshape(1, 1, C)

    full = lambda shape: pl.BlockSpec(shape, lambda b, nc: tuple(0 for _ in shape))
    out = pl.pallas_call(
        body,
        grid=(B, NC),
        in_specs=[
            pl.BlockSpec((1, BN, C), lambda b, nc: (b, nc, 0)),
            full((1, C)), full((1, C)), full((1, C)),
            full((C, C)), full((C, C)), full((C, C)),
            full((1, C)), full((1, C)),
            full((C, C)), full((1, C)),
        ],
        out_specs=pl.BlockSpec((1, 1, C), lambda b, nc: (b, 0, 0)),
        out_shape=jax.ShapeDtypeStruct((B, 1, C), jnp.float32),
        scratch_shapes=[
            pltpu.VMEM((2 * H, C), jnp.bfloat16),
            pltpu.VMEM((H, C), jnp.float32),
            pltpu.VMEM((H, 1), jnp.float32),
            pltpu.VMEM((H, 1), jnp.float32),
            pltpu.VMEM((H, 1), jnp.float32),
            pltpu.VMEM((H, 1), jnp.float32),
            pltpu.VMEM((H, 1), jnp.float32),
        ],
        compiler_params=pltpu.CompilerParams(
            dimension_semantics=("parallel", "arbitrary"),
        ),
    )(features, cf2, qg2, qb2, Wq, Wk, Wv, kvg2, kvb2, proj_W, pb2)
    return out


# speedup vs baseline: 1.3174x; 1.3174x over previous
import jax
import jax.numpy as jnp
from jax.experimental import pallas as pl
from jax.experimental.pallas import tpu as pltpu

_H = 8
_EPS = 1e-5
_BN = 1024


def kernel(class_feature, features, q_gamma, q_beta, Wq, kv_gamma, kv_beta, Wkv, proj_W, proj_b):
    B, N, C = features.shape
    H = _H
    D = C // H
    BN = _BN
    NC = N // BN
    scale = C ** -0.5

    cf2 = class_feature.reshape(1, C)
    qg2 = q_gamma.reshape(1, C)
    qb2 = q_beta.reshape(1, C)
    kvg2 = kv_gamma.reshape(1, C)
    kvb2 = kv_beta.reshape(1, C)
    pb2 = proj_b.reshape(1, C)
    Wk = Wkv[:C]
    Wv = Wkv[C:]

    def prep_body(cf_ref, qg_ref, qb_ref, wq_ref, wk_ref, kvg_ref, kvb_ref,
                  waug_ref, gp_ref):
        head_mask = jnp.where(
            jax.lax.broadcasted_iota(jnp.int32, (H, C), 1) // D
            == jax.lax.broadcasted_iota(jnp.int32, (H, C), 0),
            1.0, 0.0).astype(jnp.float32)
        cf = cf_ref[...]
        mu = jnp.mean(cf, axis=1, keepdims=True)
        xc = cf - mu
        var = jnp.mean(xc * xc, axis=1, keepdims=True)
        ln = xc * jax.lax.rsqrt(var + _EPS) * qg_ref[...] + qb_ref[...]
        q = jax.lax.dot_general(ln, wq_ref[...], (((1,), (1,)), ((), ())),
                                preferred_element_type=jnp.float32)
        qs = q * scale
        A = jnp.broadcast_to(qs, (H, C)) * head_mask
        wl = jax.lax.dot_general(A, wk_ref[...], (((1,), (0,)), ((), ())),
                                 preferred_element_type=jnp.float32)
        wlg = wl * kvg_ref[...]
        waug_ref[0:H, :] = wlg.astype(jnp.bfloat16)
        row = jax.lax.broadcasted_iota(jnp.int32, (8, C), 0)
        waug_ref[H:2 * H, :] = jnp.where(row == 0, 1.0, 0.0).astype(jnp.bfloat16)
        g1 = jnp.sum(wlg, axis=1, keepdims=True)
        g0 = jnp.sum(wl * kvb_ref[...], axis=1, keepdims=True)
        lane = jax.lax.broadcasted_iota(jnp.int32, (H, 128), 1)
        gp_ref[...] = jnp.where(lane == 0, g0, jnp.where(lane == 1, g1, 0.0))

    waug, gpair = pl.pallas_call(
        prep_body,
        out_shape=(jax.ShapeDtypeStruct((2 * H, C), jnp.bfloat16),
                   jax.ShapeDtypeStruct((H, 128), jnp.float32)),
    )(cf2, qg2, qb2, Wq, Wk, kvg2, kvb2)

    def body(x_ref, waug_ref, gp_ref, cf_ref, kvg_ref, kvb_ref, wv_ref, pw_ref,
             pb_ref, o_ref, acc, m_s, d_s, t_s):
        nc = pl.program_id(1)

        @pl.when(nc == 0)
        def _init():
            m_s[...] = jnp.full((H, 1), -1e30, jnp.float32)
            d_s[...] = jnp.zeros((H, 1), jnp.float32)
            t_s[...] = jnp.zeros((H, 1), jnp.float32)
            acc[...] = jnp.zeros((H, C), jnp.float32)

        g0 = gp_ref[:, 0:1]
        g1 = gp_ref[:, 1:2]

        x = x_ref[0]
        xb = x.astype(jnp.bfloat16)
        m1 = jax.lax.dot_general(waug_ref[...], xb, (((1,), (1,)), ((), ())),
                                 preferred_element_type=jnp.float32)
        ones_row = jnp.ones((1, C), jnp.bfloat16)
        m2 = jax.lax.dot_general(ones_row, xb * xb, (((1,), (1,)), ((), ())),
                                 preferred_element_type=jnp.float32)

        inv_c = 1.0 / C
        mu_r = m1[H:H + 1, :] * inv_c
        var_r = m2 * inv_c - mu_r * mu_r
        s_r = jax.lax.rsqrt(var_r + _EPS)
        logits = s_r * (m1[0:H, :] - mu_r * g1) + g0

        m_prev = m_s[...]
        lm = jnp.max(logits, axis=1, keepdims=True)
        m_new = jnp.maximum(m_prev, lm)
        alpha = jnp.exp(m_prev - m_new)
        p = jnp.exp(logits - m_new)
        ps = p * s_r
        d_s[...] = d_s[...] * alpha + jnp.sum(p, axis=1, keepdims=True)
        t_s[...] = t_s[...] * alpha + jnp.sum(ps * mu_r, axis=1, keepdims=True)
        m_s[...] = m_new
        acc[...] = acc[...] * alpha + jax.lax.dot_general(
            ps.astype(jnp.bfloat16), xb, (((1,), (0,)), ((), ())),
            preferred_element_type=jnp.float32)

        @pl.when(nc == NC - 1)
        def _fin():
            head_mask = jnp.where(
                jax.lax.broadcasted_iota(jnp.int32, (H, C), 1) // D
                == jax.lax.broadcasted_iota(jnp.int32, (H, C), 0),
                1.0, 0.0).astype(jnp.float32)
            dinv = 1.0 / d_s[...]
            S = kvg_ref[...] * (acc[...] * dinv - t_s[...] * dinv) + kvb_ref[...]
            R = jax.lax.dot_general(S, wv_ref[...], (((1,), (1,)), ((), ())),
                                    preferred_element_type=jnp.float32)
            agg = jnp.sum(R * head_mask, axis=0, keepdims=True)
            o = jax.lax.dot_general(agg, pw_ref[...], (((1,), (1,)), ((), ())),
                                    preferred_element_type=jnp.float32)
            o_ref[...] = (cf_ref[...] + o + pb_ref[...]).reshape(1, 1, C)

    full = lambda shape: pl.BlockSpec(shape, lambda b, nc: tuple(0 for _ in shape))
    out = pl.pallas_call(
        body,
        grid=(B, NC),
        in_specs=[
            pl.BlockSpec((1, BN, C), lambda b, nc: (b, nc, 0)),
            full((2 * H, C)), full((H, 128)),
            full((1, C)), full((1, C)), full((1, C)),
            full((C, C)), full((C, C)), full((1, C)),
        ],
        out_specs=pl.BlockSpec((1, 1, C), lambda b, nc: (b, 0, 0)),
        out_shape=jax.ShapeDtypeStruct((B, 1, C), jnp.float32),
        scratch_shapes=[
            pltpu.VMEM((H, C), jnp.float32),
            pltpu.VMEM((H, 1), jnp.float32),
            pltpu.VMEM((H, 1), jnp.float32),
            pltpu.VMEM((H, 1), jnp.float32),
        ],
        compiler_params=pltpu.CompilerParams(
            dimension_semantics=("parallel", "arbitrary"),
        ),
    )(features, waug, gpair, cf2, kvg2, kvb2, Wv, proj_W, pb2)
    return out


# speedup vs baseline: 1.4379x; 1.0915x over previous
import jax
import jax.numpy as jnp
from jax.experimental import pallas as pl
from jax.experimental.pallas import tpu as pltpu

_H = 8
_EPS = 1e-5
_BN = 1024


def kernel(class_feature, features, q_gamma, q_beta, Wq, kv_gamma, kv_beta, Wkv, proj_W, proj_b):
    B, N, C = features.shape
    H = _H
    D = C // H
    BN = _BN
    NC = N // BN
    scale = C ** -0.5

    cf2 = class_feature.reshape(1, C)
    qg2 = q_gamma.reshape(1, C)
    qb2 = q_beta.reshape(1, C)
    kvg2 = kv_gamma.reshape(1, C)
    kvb2 = kv_beta.reshape(1, C)
    pb2 = proj_b.reshape(1, C)
    Wk = Wkv[:C]
    Wv_bf = Wkv[C:].astype(jnp.bfloat16)
    pw_bf = proj_W.astype(jnp.bfloat16)

    def prep_body(cf_ref, qg_ref, qb_ref, wq_ref, wk_ref, kvg_ref, kvb_ref,
                  waug_ref, gp_ref):
        head_mask = jnp.where(
            jax.lax.broadcasted_iota(jnp.int32, (H, C), 1) // D
            == jax.lax.broadcasted_iota(jnp.int32, (H, C), 0),
            1.0, 0.0).astype(jnp.float32)
        cf = cf_ref[...]
        mu = jnp.mean(cf, axis=1, keepdims=True)
        xc = cf - mu
        var = jnp.mean(xc * xc, axis=1, keepdims=True)
        ln = xc * jax.lax.rsqrt(var + _EPS) * qg_ref[...] + qb_ref[...]
        q = jax.lax.dot_general(ln, wq_ref[...], (((1,), (1,)), ((), ())),
                                preferred_element_type=jnp.float32)
        qs = q * scale
        A = jnp.broadcast_to(qs, (H, C)) * head_mask
        wl = jax.lax.dot_general(A, wk_ref[...], (((1,), (0,)), ((), ())),
                                 preferred_element_type=jnp.float32)
        wlg = wl * kvg_ref[...]
        waug_ref[0:H, :] = wlg.astype(jnp.bfloat16)
        row = jax.lax.broadcasted_iota(jnp.int32, (8, C), 0)
        waug_ref[H:2 * H, :] = jnp.where(row == 0, 1.0, 0.0).astype(jnp.bfloat16)
        g1 = jnp.sum(wlg, axis=1, keepdims=True)
        g0 = jnp.sum(wl * kvb_ref[...], axis=1, keepdims=True)
        lane = jax.lax.broadcasted_iota(jnp.int32, (H, 128), 1)
        gp_ref[...] = jnp.where(lane == 0, g0, jnp.where(lane == 1, g1, 0.0))

    waug, gpair = pl.pallas_call(
        prep_body,
        out_shape=(jax.ShapeDtypeStruct((2 * H, C), jnp.bfloat16),
                   jax.ShapeDtypeStruct((H, 128), jnp.float32)),
    )(cf2, qg2, qb2, Wq, Wk, kvg2, kvb2)

    def body(x_ref, waug_ref, gp_ref, kvg_ref, kvb_ref,
             s_ref, acc, m_s, d_s, t_s):
        nc = pl.program_id(1)

        @pl.when(nc == 0)
        def _init():
            m_s[...] = jnp.full((H, 1), -1e30, jnp.float32)
            d_s[...] = jnp.zeros((H, 1), jnp.float32)
            t_s[...] = jnp.zeros((H, 1), jnp.float32)
            acc[...] = jnp.zeros((H, C), jnp.float32)

        g0 = gp_ref[:, 0:1]
        g1 = gp_ref[:, 1:2]

        x = x_ref[0]
        xb = x.astype(jnp.bfloat16)
        m1 = jax.lax.dot_general(waug_ref[...], xb, (((1,), (1,)), ((), ())),
                                 preferred_element_type=jnp.float32)
        ones_row = jnp.ones((1, C), jnp.bfloat16)
        m2 = jax.lax.dot_general(ones_row, xb * xb, (((1,), (1,)), ((), ())),
                                 preferred_element_type=jnp.float32)

        inv_c = 1.0 / C
        mu_r = m1[H:H + 1, :] * inv_c
        var_r = m2 * inv_c - mu_r * mu_r
        s_r = jax.lax.rsqrt(var_r + _EPS)
        logits = s_r * (m1[0:H, :] - mu_r * g1) + g0

        m_prev = m_s[...]
        lm = jnp.max(logits, axis=1, keepdims=True)
        m_new = jnp.maximum(m_prev, lm)
        alpha = jnp.exp(m_prev - m_new)
        p = jnp.exp(logits - m_new)
        ps = p * s_r
        d_s[...] = d_s[...] * alpha + jnp.sum(p, axis=1, keepdims=True)
        t_s[...] = t_s[...] * alpha + jnp.sum(ps * mu_r, axis=1, keepdims=True)
        m_s[...] = m_new
        acc[...] = acc[...] * alpha + jax.lax.dot_general(
            ps.astype(jnp.bfloat16), xb, (((1,), (0,)), ((), ())),
            preferred_element_type=jnp.float32)

        @pl.when(nc == NC - 1)
        def _fin():
            dinv = 1.0 / d_s[...]
            S = kvg_ref[...] * (acc[...] * dinv - t_s[...] * dinv) + kvb_ref[...]
            s_ref[...] = S.reshape(1, H, C)

    full2 = lambda shape: pl.BlockSpec(shape, lambda b, nc: tuple(0 for _ in shape))
    s_all = pl.pallas_call(
        body,
        grid=(B, NC),
        in_specs=[
            pl.BlockSpec((1, BN, C), lambda b, nc: (b, nc, 0)),
            full2((2 * H, C)), full2((H, 128)),
            full2((1, C)), full2((1, C)),
        ],
        out_specs=pl.BlockSpec((1, H, C), lambda b, nc: (b, 0, 0)),
        out_shape=jax.ShapeDtypeStruct((B, H, C), jnp.float32),
        scratch_shapes=[
            pltpu.VMEM((H, C), jnp.float32),
            pltpu.VMEM((H, 1), jnp.float32),
            pltpu.VMEM((H, 1), jnp.float32),
            pltpu.VMEM((H, 1), jnp.float32),
        ],
        compiler_params=pltpu.CompilerParams(
            dimension_semantics=("parallel", "arbitrary"),
        ),
    )(features, waug, gpair, kvg2, kvb2)

    def fin_body(s_ref, wv_ref, pw_ref, cf_ref, pb_ref, o_ref):
        head_mask = jnp.where(
            jax.lax.broadcasted_iota(jnp.int32, (H, C), 1) // D
            == jax.lax.broadcasted_iota(jnp.int32, (H, C), 0),
            1.0, 0.0).astype(jnp.float32)
        Sb = s_ref[...].astype(jnp.bfloat16)
        R = jax.lax.dot_general(Sb, wv_ref[...], (((1,), (1,)), ((), ())),
                                preferred_element_type=jnp.float32)
        agg = jnp.sum(R.reshape(B, H, C) * head_mask[None], axis=1)
        o = jax.lax.dot_general(agg.astype(jnp.bfloat16), pw_ref[...],
                                (((1,), (1,)), ((), ())),
                                preferred_element_type=jnp.float32)
        o_ref[...] = (cf_ref[...] + o + pb_ref[...]).reshape(B, 1, C)

    out = pl.pallas_call(
        fin_body,
        out_shape=jax.ShapeDtypeStruct((B, 1, C), jnp.float32),
    )(s_all.reshape(B * H, C), Wv_bf, pw_bf, cf2, pb2)
    return out


# speedup vs baseline: 1.6537x; 1.1501x over previous
import jax
import jax.numpy as jnp
from jax.experimental import pallas as pl
from jax.experimental.pallas import tpu as pltpu

_H = 8
_EPS = 1e-5
_BN = 2048


def kernel(class_feature, features, q_gamma, q_beta, Wq, kv_gamma, kv_beta, Wkv, proj_W, proj_b):
    B, N, C = features.shape
    H = _H
    D = C // H
    BN = _BN
    NC = N // BN
    scale = C ** -0.5

    cf2 = class_feature.reshape(1, C)
    qg2 = q_gamma.reshape(1, C)
    qb2 = q_beta.reshape(1, C)
    kvg2 = kv_gamma.reshape(1, C)
    kvb2 = kv_beta.reshape(1, C)
    pb2 = proj_b.reshape(1, C)
    Wk = Wkv[:C]
    Wv_bf = Wkv[C:].astype(jnp.bfloat16)
    pw_bf = proj_W.astype(jnp.bfloat16)

    def prep_body(cf_ref, qg_ref, qb_ref, wq_ref, wk_ref, kvg_ref, kvb_ref,
                  waug_ref, gp_ref):
        head_mask = jnp.where(
            jax.lax.broadcasted_iota(jnp.int32, (H, C), 1) // D
            == jax.lax.broadcasted_iota(jnp.int32, (H, C), 0),
            1.0, 0.0).astype(jnp.float32)
        cf = cf_ref[...]
        mu = jnp.mean(cf, axis=1, keepdims=True)
        xc = cf - mu
        var = jnp.mean(xc * xc, axis=1, keepdims=True)
        ln = xc * jax.lax.rsqrt(var + _EPS) * qg_ref[...] + qb_ref[...]
        q = jax.lax.dot_general(ln, wq_ref[...], (((1,), (1,)), ((), ())),
                                preferred_element_type=jnp.float32)
        qs = q * scale
        A = jnp.broadcast_to(qs, (H, C)) * head_mask
        wl = jax.lax.dot_general(A, wk_ref[...], (((1,), (0,)), ((), ())),
                                 preferred_element_type=jnp.float32)
        wlg = wl * kvg_ref[...]
        waug_ref[0:H, :] = wlg.astype(jnp.bfloat16)
        row = jax.lax.broadcasted_iota(jnp.int32, (8, C), 0)
        waug_ref[H:2 * H, :] = jnp.where(row == 0, 1.0, 0.0).astype(jnp.bfloat16)
        g1 = jnp.sum(wlg, axis=1, keepdims=True)
        g0 = jnp.sum(wl * kvb_ref[...], axis=1, keepdims=True)
        lane = jax.lax.broadcasted_iota(jnp.int32, (H, 128), 1)
        gp_ref[...] = jnp.where(lane == 0, g0, jnp.where(lane == 1, g1, 0.0))

    waug, gpair = pl.pallas_call(
        prep_body,
        out_shape=(jax.ShapeDtypeStruct((2 * H, C), jnp.bfloat16),
                   jax.ShapeDtypeStruct((H, 128), jnp.float32)),
    )(cf2, qg2, qb2, Wq, Wk, kvg2, kvb2)

    def body(x_ref, waug_ref, gp_ref, kvg_ref, kvb_ref,
             s_ref, acc, m_s, d_s, t_s):
        nc = pl.program_id(1)

        @pl.when(nc == 0)
        def _init():
            m_s[...] = jnp.full((H, 1), -1e30, jnp.float32)
            d_s[...] = jnp.zeros((H, 1), jnp.float32)
            t_s[...] = jnp.zeros((H, 1), jnp.float32)
            acc[...] = jnp.zeros((H, C), jnp.float32)

        g0 = gp_ref[:, 0:1]
        g1 = gp_ref[:, 1:2]

        x = x_ref[0]
        xb = x.astype(jnp.bfloat16)
        m1 = jax.lax.dot_general(waug_ref[...], xb, (((1,), (1,)), ((), ())),
                                 preferred_element_type=jnp.float32)
        ones_row = jnp.ones((1, C), jnp.bfloat16)
        m2 = jax.lax.dot_general(ones_row, xb * xb, (((1,), (1,)), ((), ())),
                                 preferred_element_type=jnp.float32)

        inv_c = 1.0 / C
        mu_r = m1[H:H + 1, :] * inv_c
        var_r = m2 * inv_c - mu_r * mu_r
        s_r = jax.lax.rsqrt(var_r + _EPS)
        logits = s_r * (m1[0:H, :] - mu_r * g1) + g0

        m_prev = m_s[...]
        lm = jnp.max(logits, axis=1, keepdims=True)
        m_new = jnp.maximum(m_prev, lm)
        alpha = jnp.exp(m_prev - m_new)
        p = jnp.exp(logits - m_new)
        ps = p * s_r
        d_s[...] = d_s[...] * alpha + jnp.sum(p, axis=1, keepdims=True)
        t_s[...] = t_s[...] * alpha + jnp.sum(ps * mu_r, axis=1, keepdims=True)
        m_s[...] = m_new
        acc[...] = acc[...] * alpha + jax.lax.dot_general(
            ps.astype(jnp.bfloat16), xb, (((1,), (0,)), ((), ())),
            preferred_element_type=jnp.float32)

        @pl.when(nc == NC - 1)
        def _fin():
            dinv = 1.0 / d_s[...]
            S = kvg_ref[...] * (acc[...] * dinv - t_s[...] * dinv) + kvb_ref[...]
            s_ref[...] = S.reshape(1, H, C)

    full2 = lambda shape: pl.BlockSpec(shape, lambda b, nc: tuple(0 for _ in shape))
    s_all = pl.pallas_call(
        body,
        grid=(B, NC),
        in_specs=[
            pl.BlockSpec((1, BN, C), lambda b, nc: (b, nc, 0)),
            full2((2 * H, C)), full2((H, 128)),
            full2((1, C)), full2((1, C)),
        ],
        out_specs=pl.BlockSpec((1, H, C), lambda b, nc: (b, 0, 0)),
        out_shape=jax.ShapeDtypeStruct((B, H, C), jnp.float32),
        scratch_shapes=[
            pltpu.VMEM((H, C), jnp.float32),
            pltpu.VMEM((H, 1), jnp.float32),
            pltpu.VMEM((H, 1), jnp.float32),
            pltpu.VMEM((H, 1), jnp.float32),
        ],
        compiler_params=pltpu.CompilerParams(
            dimension_semantics=("parallel", "arbitrary"),
        ),
    )(features, waug, gpair, kvg2, kvb2)

    def fin_body(s_ref, wv_ref, pw_ref, cf_ref, pb_ref, o_ref):
        head_mask = jnp.where(
            jax.lax.broadcasted_iota(jnp.int32, (H, C), 1) // D
            == jax.lax.broadcasted_iota(jnp.int32, (H, C), 0),
            1.0, 0.0).astype(jnp.float32)
        Sb = s_ref[...].astype(jnp.bfloat16)
        R = jax.lax.dot_general(Sb, wv_ref[...], (((1,), (1,)), ((), ())),
                                preferred_element_type=jnp.float32)
        agg = jnp.sum(R.reshape(B, H, C) * head_mask[None], axis=1)
        o = jax.lax.dot_general(agg.astype(jnp.bfloat16), pw_ref[...],
                                (((1,), (1,)), ((), ())),
                                preferred_element_type=jnp.float32)
        o_ref[...] = (cf_ref[...] + o + pb_ref[...]).reshape(B, 1, C)

    out = pl.pallas_call(
        fin_body,
        out_shape=jax.ShapeDtypeStruct((B, 1, C), jnp.float32),
    )(s_all.reshape(B * H, C), Wv_bf, pw_bf, cf2, pb2)
    return out


# speedup vs baseline: 1.7869x; 1.0805x over previous
import jax
import jax.numpy as jnp
from jax.experimental import pallas as pl
from jax.experimental.pallas import tpu as pltpu

_H = 8
_EPS = 1e-5
_BN = 4096


def kernel(class_feature, features, q_gamma, q_beta, Wq, kv_gamma, kv_beta, Wkv, proj_W, proj_b):
    B, N, C = features.shape
    H = _H
    D = C // H
    BN = _BN
    NC = N // BN
    scale = C ** -0.5

    cf2 = class_feature.reshape(1, C)
    qg2 = q_gamma.reshape(1, C)
    qb2 = q_beta.reshape(1, C)
    kvg2 = kv_gamma.reshape(1, C)
    kvb2 = kv_beta.reshape(1, C)
    pb2 = proj_b.reshape(1, C)
    Wk = Wkv[:C]
    Wv_bf = Wkv[C:].astype(jnp.bfloat16)
    pw_bf = proj_W.astype(jnp.bfloat16)

    def prep_body(cf_ref, qg_ref, qb_ref, wq_ref, wk_ref, kvg_ref, kvb_ref,
                  waug_ref, gp_ref):
        head_mask = jnp.where(
            jax.lax.broadcasted_iota(jnp.int32, (H, C), 1) // D
            == jax.lax.broadcasted_iota(jnp.int32, (H, C), 0),
            1.0, 0.0).astype(jnp.float32)
        cf = cf_ref[...]
        mu = jnp.mean(cf, axis=1, keepdims=True)
        xc = cf - mu
        var = jnp.mean(xc * xc, axis=1, keepdims=True)
        ln = xc * jax.lax.rsqrt(var + _EPS) * qg_ref[...] + qb_ref[...]
        q = jax.lax.dot_general(ln, wq_ref[...], (((1,), (1,)), ((), ())),
                                preferred_element_type=jnp.float32)
        qs = q * scale
        A = jnp.broadcast_to(qs, (H, C)) * head_mask
        wl = jax.lax.dot_general(A, wk_ref[...], (((1,), (0,)), ((), ())),
                                 preferred_element_type=jnp.float32)
        wlg = wl * kvg_ref[...]
        waug_ref[0:H, :] = wlg.astype(jnp.bfloat16)
        row = jax.lax.broadcasted_iota(jnp.int32, (8, C), 0)
        waug_ref[H:2 * H, :] = jnp.where(row == 0, 1.0, 0.0).astype(jnp.bfloat16)
        g1 = jnp.sum(wlg, axis=1, keepdims=True)
        g0 = jnp.sum(wl * kvb_ref[...], axis=1, keepdims=True)
        lane = jax.lax.broadcasted_iota(jnp.int32, (H, 128), 1)
        gp_ref[...] = jnp.where(lane == 0, g0, jnp.where(lane == 1, g1, 0.0))

    waug, gpair = pl.pallas_call(
        prep_body,
        out_shape=(jax.ShapeDtypeStruct((2 * H, C), jnp.bfloat16),
                   jax.ShapeDtypeStruct((H, 128), jnp.float32)),
    )(cf2, qg2, qb2, Wq, Wk, kvg2, kvb2)

    def body(x_ref, waug_ref, gp_ref, kvg_ref, kvb_ref,
             s_ref, acc, m_s, d_s, t_s):
        nc = pl.program_id(1)

        @pl.when(nc == 0)
        def _init():
            m_s[...] = jnp.full((H, 1), -1e30, jnp.float32)
            d_s[...] = jnp.zeros((H, 1), jnp.float32)
            t_s[...] = jnp.zeros((H, 1), jnp.float32)
            acc[...] = jnp.zeros((H, C), jnp.float32)

        g0 = gp_ref[:, 0:1]
        g1 = gp_ref[:, 1:2]

        x = x_ref[0]
        xb = x.astype(jnp.bfloat16)
        m1 = jax.lax.dot_general(waug_ref[...], xb, (((1,), (1,)), ((), ())),
                                 preferred_element_type=jnp.float32)
        ones_row = jnp.ones((1, C), jnp.bfloat16)
        m2 = jax.lax.dot_general(ones_row, xb * xb, (((1,), (1,)), ((), ())),
                                 preferred_element_type=jnp.float32)

        inv_c = 1.0 / C
        mu_r = m1[H:H + 1, :] * inv_c
        var_r = m2 * inv_c - mu_r * mu_r
        s_r = jax.lax.rsqrt(var_r + _EPS)
        logits = s_r * (m1[0:H, :] - mu_r * g1) + g0

        m_prev = m_s[...]
        lm = jnp.max(logits, axis=1, keepdims=True)
        m_new = jnp.maximum(m_prev, lm)
        alpha = jnp.exp(m_prev - m_new)
        p = jnp.exp(logits - m_new)
        ps = p * s_r
        d_s[...] = d_s[...] * alpha + jnp.sum(p, axis=1, keepdims=True)
        t_s[...] = t_s[...] * alpha + jnp.sum(ps * mu_r, axis=1, keepdims=True)
        m_s[...] = m_new
        acc[...] = acc[...] * alpha + jax.lax.dot_general(
            ps.astype(jnp.bfloat16), xb, (((1,), (0,)), ((), ())),
            preferred_element_type=jnp.float32)

        @pl.when(nc == NC - 1)
        def _fin():
            dinv = 1.0 / d_s[...]
            S = kvg_ref[...] * (acc[...] * dinv - t_s[...] * dinv) + kvb_ref[...]
            s_ref[...] = S.reshape(1, H, C)

    full2 = lambda shape: pl.BlockSpec(shape, lambda b, nc: tuple(0 for _ in shape))
    s_all = pl.pallas_call(
        body,
        grid=(B, NC),
        in_specs=[
            pl.BlockSpec((1, BN, C), lambda b, nc: (b, nc, 0)),
            full2((2 * H, C)), full2((H, 128)),
            full2((1, C)), full2((1, C)),
        ],
        out_specs=pl.BlockSpec((1, H, C), lambda b, nc: (b, 0, 0)),
        out_shape=jax.ShapeDtypeStruct((B, H, C), jnp.float32),
        scratch_shapes=[
            pltpu.VMEM((H, C), jnp.float32),
            pltpu.VMEM((H, 1), jnp.float32),
            pltpu.VMEM((H, 1), jnp.float32),
            pltpu.VMEM((H, 1), jnp.float32),
        ],
        compiler_params=pltpu.CompilerParams(
            dimension_semantics=("parallel", "arbitrary"),
        ),
    )(features, waug, gpair, kvg2, kvb2)

    def fin_body(s_ref, wv_ref, pw_ref, cf_ref, pb_ref, o_ref):
        head_mask = jnp.where(
            jax.lax.broadcasted_iota(jnp.int32, (H, C), 1) // D
            == jax.lax.broadcasted_iota(jnp.int32, (H, C), 0),
            1.0, 0.0).astype(jnp.float32)
        Sb = s_ref[...].astype(jnp.bfloat16)
        R = jax.lax.dot_general(Sb, wv_ref[...], (((1,), (1,)), ((), ())),
                                preferred_element_type=jnp.float32)
        agg = jnp.sum(R.reshape(B, H, C) * head_mask[None], axis=1)
        o = jax.lax.dot_general(agg.astype(jnp.bfloat16), pw_ref[...],
                                (((1,), (1,)), ((), ())),
                                preferred_element_type=jnp.float32)
        o_ref[...] = (cf_ref[...] + o + pb_ref[...]).reshape(B, 1, C)

    out = pl.pallas_call(
        fin_body,
        out_shape=jax.ShapeDtypeStruct((B, 1, C), jnp.float32),
    )(s_all.reshape(B * H, C), Wv_bf, pw_bf, cf2, pb2)
    return out


# speedup vs baseline: 1.7891x; 1.0012x over previous
import jax
import jax.numpy as jnp
from jax.experimental import pallas as pl
from jax.experimental.pallas import tpu as pltpu

_H = 8
_EPS = 1e-5
_BN = 4096


def kernel(class_feature, features, q_gamma, q_beta, Wq, kv_gamma, kv_beta, Wkv, proj_W, proj_b):
    B, N, C = features.shape
    H = _H
    D = C // H
    BN = _BN
    NC = N // BN
    scale = C ** -0.5

    cf2 = class_feature.reshape(1, C)
    qg2 = q_gamma.reshape(1, C)
    qb2 = q_beta.reshape(1, C)
    kvg2 = kv_gamma.reshape(1, C)
    kvb2 = kv_beta.reshape(1, C)
    pb2 = proj_b.reshape(1, C)
    Wk = Wkv[:C]
    Wv_bf = Wkv[C:].astype(jnp.bfloat16)
    pw_bf = proj_W.astype(jnp.bfloat16)

    def prep_body(cf_ref, qg_ref, qb_ref, wq_ref, wk_ref, kvg_ref, kvb_ref,
                  waug_ref, gp_ref):
        head_mask = jnp.where(
            jax.lax.broadcasted_iota(jnp.int32, (H, C), 1) // D
            == jax.lax.broadcasted_iota(jnp.int32, (H, C), 0),
            1.0, 0.0).astype(jnp.float32)
        cf = cf_ref[...]
        mu = jnp.mean(cf, axis=1, keepdims=True)
        xc = cf - mu
        var = jnp.mean(xc * xc, axis=1, keepdims=True)
        ln = xc * jax.lax.rsqrt(var + _EPS) * qg_ref[...] + qb_ref[...]
        q = jax.lax.dot_general(ln, wq_ref[...], (((1,), (1,)), ((), ())),
                                preferred_element_type=jnp.float32)
        qs = q * scale
        A = jnp.broadcast_to(qs, (H, C)) * head_mask
        wl = jax.lax.dot_general(A, wk_ref[...], (((1,), (0,)), ((), ())),
                                 preferred_element_type=jnp.float32)
        wlg = wl * kvg_ref[...]
        waug_ref[0:H, :] = wlg.astype(jnp.bfloat16)
        row = jax.lax.broadcasted_iota(jnp.int32, (8, C), 0)
        waug_ref[H:2 * H, :] = jnp.where(row == 0, 1.0, 0.0).astype(jnp.bfloat16)
        g1 = jnp.sum(wlg, axis=1, keepdims=True)
        g0 = jnp.sum(wl * kvb_ref[...], axis=1, keepdims=True)
        lane = jax.lax.broadcasted_iota(jnp.int32, (H, 128), 1)
        gp_ref[...] = jnp.where(lane == 0, g0, jnp.where(lane == 1, g1, 0.0))

    waug, gpair = pl.pallas_call(
        prep_body,
        out_shape=(jax.ShapeDtypeStruct((2 * H, C), jnp.bfloat16),
                   jax.ShapeDtypeStruct((H, 128), jnp.float32)),
    )(cf2, qg2, qb2, Wq, Wk, kvg2, kvb2)

    def body(x_ref, waug_ref, gp_ref, kvg_ref, kvb_ref,
             s_ref, acc, m_s, d_s, t_s):
        nc = pl.program_id(1)

        @pl.when(nc == 0)
        def _init():
            m_s[...] = jnp.full((H, 1), -1e30, jnp.float32)
            d_s[...] = jnp.zeros((H, 1), jnp.float32)
            t_s[...] = jnp.zeros((H, 1), jnp.float32)
            acc[...] = jnp.zeros((H, C), jnp.float32)

        g0 = gp_ref[:, 0:1]
        g1 = gp_ref[:, 1:2]

        x = x_ref[0]
        xb = x.astype(jnp.bfloat16)
        m1 = jax.lax.dot_general(waug_ref[...], xb, (((1,), (1,)), ((), ())),
                                 preferred_element_type=jnp.float32)
        ones_row = jnp.ones((1, C), jnp.bfloat16)
        m2 = jax.lax.dot_general(ones_row, xb * xb, (((1,), (1,)), ((), ())),
                                 preferred_element_type=jnp.float32)

        inv_c = 1.0 / C
        mu_r = m1[H:H + 1, :] * inv_c
        var_r = m2 * inv_c - mu_r * mu_r
        s_r = jax.lax.rsqrt(var_r + _EPS)
        logits = s_r * (m1[0:H, :] - mu_r * g1) + g0

        m_prev = m_s[...]
        lm = jnp.max(logits, axis=1, keepdims=True)
        m_new = jnp.maximum(m_prev, lm)
        alpha = jnp.exp(m_prev - m_new)
        p = jnp.exp(logits - m_new)
        ps = p * s_r
        d_s[...] = d_s[...] * alpha + jnp.sum(p, axis=1, keepdims=True)
        t_s[...] = t_s[...] * alpha + jnp.sum(ps * mu_r, axis=1, keepdims=True)
        m_s[...] = m_new
        acc[...] = acc[...] * alpha + jax.lax.dot_general(
            ps.astype(jnp.bfloat16), xb, (((1,), (0,)), ((), ())),
            preferred_element_type=jnp.float32)

        @pl.when(nc == NC - 1)
        def _fin():
            dinv = 1.0 / d_s[...]
            S = kvg_ref[...] * (acc[...] * dinv - t_s[...] * dinv) + kvb_ref[...]
            s_ref[...] = S.reshape(1, H, C)

    full2 = lambda shape: pl.BlockSpec(shape, lambda b, nc: tuple(0 for _ in shape))
    s_all = pl.pallas_call(
        body,
        grid=(B, NC),
        in_specs=[
            pl.BlockSpec((1, BN, C), lambda b, nc: (b, nc, 0)),
            full2((2 * H, C)), full2((H, 128)),
            full2((1, C)), full2((1, C)),
        ],
        out_specs=pl.BlockSpec((1, H, C), lambda b, nc: (b, 0, 0)),
        out_shape=jax.ShapeDtypeStruct((B, H, C), jnp.float32),
        scratch_shapes=[
            pltpu.VMEM((H, C), jnp.float32),
            pltpu.VMEM((H, 1), jnp.float32),
            pltpu.VMEM((H, 1), jnp.float32),
            pltpu.VMEM((H, 1), jnp.float32),
        ],
        compiler_params=pltpu.CompilerParams(
            dimension_semantics=("arbitrary", "arbitrary"),
        ),
    )(features, waug, gpair, kvg2, kvb2)

    def fin_body(s_ref, wv_ref, pw_ref, cf_ref, pb_ref, o_ref):
        head_mask = jnp.where(
            jax.lax.broadcasted_iota(jnp.int32, (H, C), 1) // D
            == jax.lax.broadcasted_iota(jnp.int32, (H, C), 0),
            1.0, 0.0).astype(jnp.float32)
        Sb = s_ref[...].astype(jnp.bfloat16)
        R = jax.lax.dot_general(Sb, wv_ref[...], (((1,), (1,)), ((), ())),
                                preferred_element_type=jnp.float32)
        agg = jnp.sum(R.reshape(B, H, C) * head_mask[None], axis=1)
        o = jax.lax.dot_general(agg.astype(jnp.bfloat16), pw_ref[...],
                                (((1,), (1,)), ((), ())),
                                preferred_element_type=jnp.float32)
        o_ref[...] = (cf_ref[...] + o + pb_ref[...]).reshape(B, 1, C)

    out = pl.pallas_call(
        fin_body,
        out_shape=jax.ShapeDtypeStruct((B, 1, C), jnp.float32),
    )(s_all.reshape(B * H, C), Wv_bf, pw_bf, cf2, pb2)
    return out
